# baseline (speedup 1.0000x reference)
import jax
import jax.numpy as jnp
from jax import lax
from jax.experimental import pallas as pl
from jax.experimental.pallas import tpu as pltpu
from jax.experimental.pallas import tpu_sc as plsc

N = 10000
E = 320000
HID = 128
NUM_CLASS = 64

NC, NS, L = 2, 16, 16
NW = NC * NS
EPT = E // NW
CH = 128
NCHUNK = 80
EPP = NCHUNK * CH
SUP = 8
NBUF = 2
NSUP = NCHUNK // SUP
NPAD = 10112
NPT = NPAD // NS
NDEN = 10240
ZCHUNKS = [(o, min(CH, NPT - o)) for o in range(0, NPT, CH)]

BL = 400
GRID = N // BL


def _full(shape):
    return pl.BlockSpec(shape, lambda i: tuple(0 for _ in shape))


def _rows(shape):
    return pl.BlockSpec(shape, lambda i: (i,) + tuple(0 for _ in shape[1:]))


def _pack_pairs(h):
    hb = h.astype(jnp.bfloat16)
    lo = lax.bitcast_convert_type(hb[:, :HID // 2], jnp.uint16).astype(jnp.uint32)
    hi = lax.bitcast_convert_type(hb[:, HID // 2:], jnp.uint16).astype(jnp.uint32)
    return lax.bitcast_convert_type(lo | (hi << 16), jnp.int32)


def _tc_pre_body(x_ref, w_ref, b_ref, ap_ref, h_ref, hb_ref, a_ref):
    h = jnp.dot(x_ref[...], w_ref[...], preferred_element_type=jnp.float32)
    h = h + b_ref[...]
    h_ref[...] = h
    hb_ref[...] = _pack_pairs(h)
    a_ref[...] = jnp.dot(h, ap_ref[...], preferred_element_type=jnp.float32)


def _tc_pre(x, w, b2, ap):
    return pl.pallas_call(
        _tc_pre_body,
        grid=(GRID,),
        in_specs=[_rows((BL, HID)), _full((HID, HID)), _full((1, HID)),
                  _full((HID, 8))],
        out_specs=[_rows((BL, HID)), _rows((BL, HID // 2)), _rows((BL, 8))],
        out_shape=[jax.ShapeDtypeStruct((N, HID), jnp.float32),
                   jax.ShapeDtypeStruct((NPAD, HID // 2), jnp.int32),
                   jax.ShapeDtypeStruct((N, 8), jnp.float32)],
    )(x, w, b2, ap)


def _node_update(h_ref, hp0_ref, hp1_ref, dt_ref, wap_ref, wbp_ref, gb_ref):
    d = jnp.sum(dt_ref[...], axis=1, keepdims=True) + 1e-16
    hagg = (hp0_ref[...] + hp1_ref[...]) / d
    t = (jnp.dot(h_ref[...], wap_ref[...], preferred_element_type=jnp.float32)
         + jnp.dot(hagg, wbp_ref[...], preferred_element_type=jnp.float32)
         + gb_ref[...])
    return jnp.maximum(t, 0.0)


def _tc_mid_body(h_ref, hp0_ref, hp1_ref, dt_ref, wap_ref, wbp_ref, gb_ref,
                 fw_ref, fb_ref, ap_ref, hn_ref, hb_ref, an_ref):
    t = _node_update(h_ref, hp0_ref, hp1_ref, dt_ref, wap_ref, wbp_ref, gb_ref)
    hn = jnp.dot(t, fw_ref[...], preferred_element_type=jnp.float32) + fb_ref[...]
    hn_ref[...] = hn
    hb_ref[...] = _pack_pairs(hn)
    an_ref[...] = jnp.dot(hn, ap_ref[...], preferred_element_type=jnp.float32)


def _tc_mid(h, hp0, hp1, dt, wap, wbp, gb2, fw, fb2, ap):
    return pl.pallas_call(
        _tc_mid_body,
        grid=(GRID,),
        in_specs=[_rows((BL, HID)), _rows((BL, HID)), _rows((BL, HID)),
                  _rows((BL, NC)), _full((HID, HID)), _full((HID, HID)),
                  _full((1, HID)), _full((HID, HID)), _full((1, HID)),
                  _full((HID, 8))],
        out_specs=[_rows((BL, HID)), _rows((BL, HID // 2)), _rows((BL, 8))],
        out_shape=[jax.ShapeDtypeStruct((N, HID), jnp.float32),
                   jax.ShapeDtypeStruct((NPAD, HID // 2), jnp.int32),
                   jax.ShapeDtypeStruct((N, 8), jnp.float32)],
    )(h, hp0, hp1, dt, wap, wbp, gb2, fw, fb2, ap)


def _tc_post_body(h_ref, hp0_ref, hp1_ref, dt_ref, wap_ref, wbp_ref, gb_ref,
                  fcw_ref, fcb_ref, o_ref):
    t = _node_update(h_ref, hp0_ref, hp1_ref, dt_ref, wap_ref, wbp_ref, gb_ref)
    o_ref[...] = (jnp.dot(t, fcw_ref[...], preferred_element_type=jnp.float32)
                  + fcb_ref[...])


def _tc_post(h, hp0, hp1, dt, wap, wbp, gb2, fcw, fcb2):
    return pl.pallas_call(
        _tc_post_body,
        grid=(GRID,),
        in_specs=[_rows((BL, HID)), _rows((BL, HID)), _rows((BL, HID)),
                  _rows((BL, NC)), _full((HID, HID)), _full((HID, HID)),
                  _full((1, HID)), _full((HID, NUM_CLASS)),
                  _full((1, NUM_CLASS))],
        out_specs=[_rows((BL, NUM_CLASS))],
        out_shape=[jax.ShapeDtypeStruct((N, NUM_CLASS), jnp.float32)],
    )(h, hp0, hp1, dt, wap, wbp, gb2, fcw, fcb2)[0]


def _sc_body(hb_hbm, pk_hbm, zeros_hbm, ei_hbm,
             hagg_out, den_out,
             idx_v, pk_v, exv_v, rows_v, scat_v, dbuf_v,
             hagg_sh, den_sh, gsem, ssem, dsem):
    c = lax.axis_index("c")
    s = lax.axis_index("s")
    wid = s * NC + c
    zero16 = jnp.zeros((L,), jnp.float32)
    zero16i = jnp.zeros((L,), jnp.int32)
    m16 = jnp.full((L,), -65536, jnp.int32)
    s16 = jnp.full((L,), 16, jnp.int32)

    pltpu.sync_copy(pk_hbm, pk_v)

    def _zrow(j, _):
        for g in range(HID // L):
            scat_v[j, pl.ds(g * L, L)] = zero16
        return 0
    lax.fori_loop(0, CH, _zrow, 0)
    base = s * NPT
    for o, sz in ZCHUNKS:
        pltpu.sync_copy(scat_v.at[pl.ds(0, sz)],
                        hagg_sh.at[pl.ds(base + o, sz)])
    @pl.when(s == 0)
    def _():
        pltpu.sync_copy(zeros_hbm, den_sh)

    plsc.subcore_barrier()

    def _stage(w, sp):
        pltpu.sync_copy(ei_hbm.at[0, wid, pl.ds(sp * SUP, SUP)],
                        idx_v.at[w, 0])
        pltpu.sync_copy(ei_hbm.at[1, wid, pl.ds(sp * SUP, SUP)],
                        idx_v.at[w, 1])

    _stage(0, 0)
    pltpu.async_copy(hb_hbm.at[idx_v.at[0, 0, 0]], rows_v.at[0], gsem.at[0])
    pltpu.async_copy(hb_hbm.at[idx_v.at[0, 0, 1]], rows_v.at[1], gsem.at[1])

    def _pass(sp, _):
        w = lax.rem(sp, 2)
        @pl.when(sp + 1 < NSUP)
        def _():
            _stage(1 - w, sp + 1)
        for j in range(SUP):
            b = j % NBUF
            pltpu.make_async_copy(hb_hbm.at[idx_v.at[w, 0, j]],
                                  rows_v.at[b], gsem.at[b]).wait()
            if j == 0:
                @pl.when(sp > 0)
                def _():
                    pltpu.make_async_copy(
                        exv_v.at[0], den_sh.at[idx_v.at[w, 1, j]],
                        dsem).wait()
            else:
                pltpu.make_async_copy(
                    exv_v.at[0], den_sh.at[idx_v.at[w, 1, j]], dsem).wait()

            @plsc.parallel_loop(0, CH // L, 1, unroll=2)
            def _attn(g):
                lanes = lax.iota(jnp.int32, L) + g * L
                sidx = idx_v[w, 0, j, pl.ds(g * L, L)]
                didx = idx_v[w, 1, j, pl.ds(g * L, L)]
                ws = plsc.load_gather(pk_v, [sidx])
                wd = plsc.load_gather(pk_v, [didx])
                av = plsc.bitcast(lax.bitwise_and(ws, m16), jnp.float32)
                bv = plsc.bitcast(lax.shift_left(wd, s16), jnp.float32)
                e = av + bv
                e = jnp.maximum(e, e * 0.01)
                ex = jnp.exp(e)
                plsc.store_scatter(exv_v, [zero16i, lanes], ex)
            pltpu.async_copy(exv_v.at[0], den_sh.at[idx_v.at[w, 1, j]],
                             dsem, add=True)

            if j == 0:
                @pl.when(sp > 0)
                def _():
                    pltpu.make_async_copy(
                        scat_v, hagg_sh.at[idx_v.at[w, 1, j]], ssem).wait()
            else:
                pltpu.make_async_copy(
                    scat_v, hagg_sh.at[idx_v.at[w, 1, j]], ssem).wait()

            @plsc.parallel_loop(0, CH, 1, unroll=4)
            def _scale(j2):
                jv = jnp.full((L,), j2, jnp.int32)
                exb = plsc.load_gather(exv_v, [zero16i, jv])
                for g in range(HID // 32):
                    pw = rows_v[b, j2, pl.ds(g * L, L)]
                    lo = plsc.bitcast(lax.shift_left(pw, s16), jnp.float32)
                    hi = plsc.bitcast(lax.bitwise_and(pw, m16), jnp.float32)
                    scat_v[j2, pl.ds(g * L, L)] = lo * exb
                    scat_v[j2, pl.ds(HID // 2 + g * L, L)] = hi * exb

            pltpu.async_copy(scat_v, hagg_sh.at[idx_v.at[w, 1, j]],
                             ssem, add=True)
            if j < SUP - 2:
                pltpu.async_copy(hb_hbm.at[idx_v.at[w, 0, j + 2]],
                                 rows_v.at[b], gsem.at[b])
            else:
                @pl.when(sp + 1 < NSUP)
                def _():
                    pltpu.async_copy(
                        hb_hbm.at[idx_v.at[1 - w, 0, j - (SUP - 2)]],
                        rows_v.at[b], gsem.at[b])
        return 0
    lax.fori_loop(0, NSUP, _pass, 0)
    pltpu.make_async_copy(scat_v, hagg_sh.at[idx_v.at[0, 1, 0]], ssem).wait()
    pltpu.make_async_copy(exv_v.at[0], den_sh.at[idx_v.at[0, 1, 0]],
                          dsem).wait()

    plsc.subcore_barrier()
    db = s * (NDEN // NS)
    pltpu.sync_copy(den_sh.at[pl.ds(db, NDEN // NS)], dbuf_v)
    pltpu.sync_copy(dbuf_v, den_out.at[pl.ds(c * NDEN + db, NDEN // NS)])
    for o, sz in ZCHUNKS:
        pltpu.sync_copy(hagg_sh.at[pl.ds(base + o, sz)],
                        scat_v.at[pl.ds(0, sz)])
        pltpu.sync_copy(scat_v.at[pl.ds(0, sz)],
                        hagg_out.at[c, pl.ds(base + o, sz)])


def _sc_layer(hb, pk, zeros, ei):
    mesh = plsc.VectorSubcoreMesh(core_axis_name="c", subcore_axis_name="s",
                                  num_cores=NC, num_subcores=NS)
    k = pl.kernel(
        _sc_body,
        out_type=(jax.ShapeDtypeStruct((NC, NPAD, HID), jnp.float32),
                  jax.ShapeDtypeStruct((NC * NDEN,), jnp.float32)),
        mesh=mesh,
        scratch_types=[
            pltpu.VMEM((2, 2, SUP, CH), jnp.int32),
            pltpu.VMEM((NPAD,), jnp.int32),
            pltpu.VMEM((1, CH), jnp.float32),
            pltpu.VMEM((NBUF, CH, HID // 2), jnp.int32),
            pltpu.VMEM((CH, HID), jnp.float32),
            pltpu.VMEM((NDEN // NS,), jnp.float32),
            pltpu.VMEM_SHARED((NPAD, HID), jnp.float32),
            pltpu.VMEM_SHARED((NDEN,), jnp.float32),
            pltpu.SemaphoreType.DMA((NBUF,)),
            pltpu.SemaphoreType.DMA,
            pltpu.SemaphoreType.DMA,
        ],
        compiler_params=pltpu.CompilerParams(needs_layout_passes=False, use_tc_tiling_on_sc=False),
    )
    return k(hb, pk, zeros, ei)


def kernel(x, edge_index, feat_W0, feat_b0, attn_w0, gcn_W0, gcn_b0,
           feat_W1, feat_b1, attn_w1, gcn_W1, gcn_b1, fc_W, fc_b):
    ei = jnp.pad(edge_index.reshape(2, NW, EPT),
                 ((0, 0), (0, 0), (0, EPP - EPT)),
                 constant_values=N).reshape(2, NW, NCHUNK, CH)

    def attn_pack(aw):
        ap = jnp.stack([aw[:HID], aw[HID:]], axis=1)
        return jnp.pad(ap, ((0, 0), (0, 6)))

    def pk_pack(a):
        asrc = a[:, 0].astype(jnp.bfloat16)
        adst = a[:, 1].astype(jnp.bfloat16)
        hi = lax.bitcast_convert_type(asrc, jnp.uint16).astype(jnp.uint32) << 16
        lo = lax.bitcast_convert_type(adst, jnp.uint16).astype(jnp.uint32)
        pk = lax.bitcast_convert_type(hi | lo, jnp.int32)
        return jnp.pad(pk, (0, NPAD - N))

    def den_t(den):
        return den.reshape(NC, NDEN).T

    zeros = jnp.zeros((NDEN,), jnp.float32)

    ap0 = attn_pack(attn_w0)
    ap1 = attn_pack(attn_w1)
    fb0 = feat_b0[None, :]
    fb1 = feat_b1[None, :]
    gb0 = gcn_b0[None, :]
    gb1 = gcn_b1[None, :]
    fcb = fc_b[None, :]
    wa0 = gcn_W0[:HID] - gcn_W0[HID:]
    wb0 = gcn_W0[HID:]
    wa1 = gcn_W1[:HID] - gcn_W1[HID:]
    wb1 = gcn_W1[HID:]

    h0, hb0, a0 = _tc_pre(x, feat_W0, fb0, ap0)
    hagg0, den0 = _sc_layer(hb0, pk_pack(a0), zeros, ei)
    h1, hb1, a1 = _tc_mid(h0, hagg0[0], hagg0[1], den_t(den0),
                          wa0, wb0, gb0, feat_W1, fb1, ap1)
    hagg1, den1 = _sc_layer(hb1, pk_pack(a1), zeros, ei)
    out = _tc_post(h1, hagg1[0], hagg1[1], den_t(den1),
                   wa1, wb1, gb1, fc_W, fcb)
    return out
